# Initial kernel scaffold; baseline (speedup 1.0000x reference)
import jax
import jax.numpy as jnp
from jax.experimental import pallas as pl


def kernel(feat, edge_index, W, b):
    raise NotImplementedError("write your pallas kernel here")


# speedup vs baseline: 4.7591x; 4.7591x over previous
import functools

import jax
import jax.numpy as jnp
from jax import lax
from jax.experimental import pallas as pl
from jax.experimental.pallas import tpu as pltpu
from jax.experimental.pallas import tpu_sc as plsc

NC = 2
NS = 16
NW = NC * NS
CH = 128
NBUF = 4


def _deg_body(srcd_hbm, degp_hbm, idx_v, deg_v, *, n_vec, nslot):
    c = lax.axis_index("c")
    s = lax.axis_index("s")
    w = s * NC + c
    pltpu.sync_copy(srcd_hbm.at[w], idx_v)
    zeros = jnp.zeros((16,), dtype=jnp.float32)
    ones = jnp.full((16,), 1.0, dtype=jnp.float32)

    def zbody(j, carry):
        deg_v[pl.ds(j * 16, 16)] = zeros
        return carry

    lax.fori_loop(0, nslot // 16, zbody, 0)

    def body(j, carry):
        idx = idx_v[j]
        plsc.addupdate_scatter(deg_v, [idx], ones)
        return carry

    lax.fori_loop(0, n_vec, body, 0)
    pltpu.sync_copy(deg_v, degp_hbm.at[w])


def _hop_body(x2_hbm, srcg_hbm, dsts_hbm, zero_hbm, out_hbm,
              idx_s, idx_d, rows, acc, s0, s1, s2, s3,
              *, nchunk, stripe):
    c = lax.axis_index("c")
    s = lax.axis_index("s")
    pltpu.sync_copy(zero_hbm.at[pl.ds(s * stripe, stripe)],
                    acc.at[pl.ds(s * stripe, stripe)])
    pltpu.sync_copy(srcg_hbm.at[s], idx_s)
    pltpu.sync_copy(dsts_hbm.at[s], idx_d)
    plsc.subcore_barrier()

    sems = (s0, s1, s2, s3)
    xc = x2_hbm.at[c]

    def body(i, carry):
        base = i * NBUF
        cps = [pltpu.async_copy(xc.at[idx_s.at[base + b]], rows.at[b],
                                sems[b])
               for b in range(NBUF)]
        for b in range(NBUF):
            cps[b].wait()
            pltpu.sync_copy(rows.at[b], acc.at[idx_d.at[base + b]], add=True)
        return carry

    lax.fori_loop(0, nchunk // NBUF, body, 0)
    plsc.subcore_barrier()
    pltpu.sync_copy(acc.at[pl.ds(s * stripe, stripe)],
                    out_hbm.at[c, pl.ds(s * stripe, stripe)])


def _norm_body(degp_ref, out_ref):
    d = jnp.sum(degp_ref[...], axis=0)
    out_ref[...] = lax.rsqrt(jnp.maximum(d, 1.0))


def _scale_body(f_ref, n_ref, o_ref):
    o_ref[...] = (f_ref[...] * n_ref[...][:, :, None]).swapaxes(0, 1)


def _mid_body(a_ref, n_ref, h_ref, y_ref):
    nm = n_ref[...]
    a = a_ref[...]
    h_ref[...] = jnp.concatenate([a[0], a[1]], axis=1) * nm
    y_ref[...] = a * (nm * nm)[None, :, :]


def _fin_body(f_ref, h1_ref, a_ref, n_ref, w_ref, b_ref, o_ref):
    a = a_ref[...]
    h2 = jnp.concatenate([a[0], a[1]], axis=1) * n_ref[...]
    ssum = f_ref[...] + h1_ref[...] + h2
    y = jnp.dot(ssum, w_ref[...], preferred_element_type=jnp.float32)
    o_ref[...] = jnp.maximum(y + 3.0 * b_ref[...], 0.0)


def kernel(feat, edge_index, W, b):
    N, D = feat.shape
    HD = D // NC
    E = edge_index.shape[1]

    PWd = E // NW
    nvec_pad = -(-PWd // 16) * 16
    srcd = jnp.pad(edge_index[0].reshape(NW, PWd),
                   ((0, 0), (0, nvec_pad - PWd)),
                   constant_values=N).reshape(NW, nvec_pad // 16, 16)

    PWh = E // NS
    nchunk = -(-PWh // CH)
    nchunk = ((nchunk + NBUF - 1) // NBUF) * NBUF
    pad_h = nchunk * CH - PWh
    src = edge_index[0].reshape(NS, PWh)
    dst = edge_index[1].reshape(NS, PWh)
    srcg = jnp.pad(src, ((0, 0), (0, pad_h))).reshape(NS, nchunk, CH)
    dsts = jnp.pad(dst, ((0, 0), (0, pad_h)),
                   constant_values=N).reshape(NS, nchunk, CH)

    nslot = ((N + 1 + 127) // 128) * 128
    accr = -(-(N + 1) // (NS * 8)) * (NS * 8)
    stripe = accr // NS
    zero = jnp.zeros((accr, HD), dtype=jnp.float32)

    mesh = plsc.VectorSubcoreMesh(core_axis_name="c", subcore_axis_name="s")

    deg_call = pl.kernel(
        functools.partial(_deg_body, n_vec=nvec_pad // 16, nslot=nslot),
        out_type=jax.ShapeDtypeStruct((NW, nslot), jnp.float32),
        mesh=mesh,
        scratch_types=[
            pltpu.VMEM((nvec_pad // 16, 16), jnp.int32),
            pltpu.VMEM((nslot,), jnp.float32),
        ],
        compiler_params=pltpu.CompilerParams(needs_layout_passes=False),
    )
    degp = deg_call(srcd)

    hop_call = pl.kernel(
        functools.partial(_hop_body, nchunk=nchunk, stripe=stripe),
        out_type=jax.ShapeDtypeStruct((NC, accr, HD), jnp.float32),
        mesh=mesh,
        scratch_types=[
            pltpu.VMEM((nchunk, CH), jnp.int32),
            pltpu.VMEM((nchunk, CH), jnp.int32),
            pltpu.VMEM((NBUF, CH, HD), jnp.float32),
            pltpu.VMEM_SHARED((accr, HD), jnp.float32),
            pltpu.SemaphoreType.DMA,
            pltpu.SemaphoreType.DMA,
            pltpu.SemaphoreType.DMA,
            pltpu.SemaphoreType.DMA,
        ],
        compiler_params=pltpu.CompilerParams(use_tc_tiling_on_sc=False),
    )

    norm2d = pl.pallas_call(
        _norm_body,
        out_shape=jax.ShapeDtypeStruct((nslot // 128, 128), jnp.float32),
    )(degp.reshape(NW, nslot // 128, 128))
    normcol = norm2d.reshape(nslot)[:N][:, None]

    R = 5
    B = N // R
    row_spec = pl.BlockSpec((B, D), lambda i: (i, 0))
    col_spec = pl.BlockSpec((B, 1), lambda i: (i, 0))
    half_in_spec = pl.BlockSpec((B, NC, HD), lambda i: (i, 0, 0))
    half_out_spec = pl.BlockSpec((NC, B, HD), lambda i: (0, i, 0))
    w_spec = pl.BlockSpec((D, D), lambda i: (0, 0))
    b_spec = pl.BlockSpec((1, D), lambda i: (0, 0))

    x1h = pl.pallas_call(
        _scale_body,
        grid=(R,),
        in_specs=[half_in_spec, col_spec],
        out_specs=half_out_spec,
        out_shape=jax.ShapeDtypeStruct((NC, N, HD), jnp.float32),
    )(feat.reshape(N, NC, HD), normcol)

    aggp1 = hop_call(x1h, srcg, dsts, zero)

    h1, y1h = pl.pallas_call(
        _mid_body,
        grid=(R,),
        in_specs=[half_out_spec, col_spec],
        out_specs=[row_spec, half_out_spec],
        out_shape=[jax.ShapeDtypeStruct((N, D), jnp.float32),
                   jax.ShapeDtypeStruct((NC, N, HD), jnp.float32)],
    )(aggp1, normcol)

    aggp2 = hop_call(y1h, srcg, dsts, zero)

    rst = pl.pallas_call(
        _fin_body,
        grid=(R,),
        in_specs=[row_spec, row_spec, half_out_spec, col_spec, w_spec,
                  b_spec],
        out_specs=row_spec,
        out_shape=jax.ShapeDtypeStruct((N, D), jnp.float32),
    )(feat, h1, aggp2, normcol, W, b.reshape(1, D))

    return rst
